# baseline (speedup 1.0000x reference)
import functools

import jax
import jax.numpy as jnp
from jax import lax
from jax.experimental import pallas as pl
from jax.experimental.pallas import tpu as pltpu
from jax.experimental.pallas import tpu_sc as plsc

SEQ_LEN = 8192
MODEL_DIM = 1024

_NC = 2
_NS = 16
_NW = _NC * _NS

_SC_ROWS = 4096
_TC_ROWS = SEQ_LEN - _SC_ROWS

_ROWS_PER_W = _SC_ROWS // _NW
_CHUNK = 16
_NSTEPS = _ROWS_PER_W // _CHUNK
_NBUF = min(7, _NSTEPS)

_mesh = plsc.VectorSubcoreMesh(core_axis_name="c", subcore_axis_name="s")


@functools.partial(
    pl.kernel,
    mesh=_mesh,
    out_type=jax.ShapeDtypeStruct((_SC_ROWS, MODEL_DIM), jnp.float32),
    scratch_types=[
        pltpu.VMEM((_NBUF, _CHUNK, MODEL_DIM), jnp.float32),
        pltpu.SemaphoreType.DMA((_NBUF,)),
        pltpu.SemaphoreType.DMA((_NBUF,)),
    ],
)
def _sc_copy(table_hbm, out_hbm, buf, sem_in, sem_out):
    wid = lax.axis_index("s") * _NC + lax.axis_index("c")
    base = wid * _ROWS_PER_W

    in_cp = [None] * _NSTEPS
    out_cp = [None] * _NSTEPS

    def start_in(step):
        b = step % _NBUF
        return pltpu.async_copy(
            table_hbm.at[pl.ds(base + step * _CHUNK, _CHUNK)],
            buf.at[b],
            sem_in.at[b],
        )

    for step in range(min(_NBUF, _NSTEPS)):
        in_cp[step] = start_in(step)

    for step in range(_NSTEPS):
        b = step % _NBUF
        in_cp[step].wait()
        out_cp[step] = pltpu.async_copy(
            buf.at[b],
            out_hbm.at[pl.ds(base + step * _CHUNK, _CHUNK)],
            sem_out.at[b],
        )
        prev = step - 1
        nxt = prev + _NBUF
        if prev >= 0 and nxt < _NSTEPS:
            out_cp[prev].wait()
            in_cp[nxt] = start_in(nxt)

    for step in range(max(0, _NSTEPS - _NBUF), _NSTEPS):
        out_cp[step].wait()


_TC_BLOCK = 512


def _tc_body(in_ref, out_ref):
    out_ref[...] = in_ref[...]


_tc_copy = pl.pallas_call(
    _tc_body,
    grid=(_TC_ROWS // _TC_BLOCK,),
    in_specs=[
        pl.BlockSpec(
            (_TC_BLOCK, MODEL_DIM), lambda i: (i + _SC_ROWS // _TC_BLOCK, 0)
        )
    ],
    out_specs=pl.BlockSpec((_TC_BLOCK, MODEL_DIM), lambda i: (i, 0)),
    out_shape=jax.ShapeDtypeStruct((_TC_ROWS, MODEL_DIM), jnp.float32),
)


def kernel(x, emb_weight):
    del x
    sc_part = _sc_copy(emb_weight)
    tc_part = _tc_copy(emb_weight)
    return jnp.concatenate([sc_part, tc_part], axis=0)


# speedup vs baseline: 1.2773x; 1.2773x over previous
import functools

import jax
import jax.numpy as jnp
from jax import lax
from jax.experimental import pallas as pl
from jax.experimental.pallas import tpu as pltpu
from jax.experimental.pallas import tpu_sc as plsc

SEQ_LEN = 8192
MODEL_DIM = 1024

_NC = 2
_NS = 16
_NW = _NC * _NS

_SC_ROWS = 2048
_TC_ROWS = SEQ_LEN - _SC_ROWS

_ROWS_PER_W = _SC_ROWS // _NW
_CHUNK = 16
_NSTEPS = _ROWS_PER_W // _CHUNK
_NBUF = min(7, _NSTEPS)

_mesh = plsc.VectorSubcoreMesh(core_axis_name="c", subcore_axis_name="s")


@functools.partial(
    pl.kernel,
    mesh=_mesh,
    out_type=jax.ShapeDtypeStruct((_SC_ROWS, MODEL_DIM), jnp.float32),
    scratch_types=[
        pltpu.VMEM((_NBUF, _CHUNK, MODEL_DIM), jnp.float32),
        pltpu.SemaphoreType.DMA((_NBUF,)),
        pltpu.SemaphoreType.DMA((_NBUF,)),
    ],
)
def _sc_copy(table_hbm, out_hbm, buf, sem_in, sem_out):
    wid = lax.axis_index("s") * _NC + lax.axis_index("c")
    base = wid * _ROWS_PER_W

    in_cp = [None] * _NSTEPS
    out_cp = [None] * _NSTEPS

    def start_in(step):
        b = step % _NBUF
        return pltpu.async_copy(
            table_hbm.at[pl.ds(base + step * _CHUNK, _CHUNK)],
            buf.at[b],
            sem_in.at[b],
        )

    for step in range(min(_NBUF, _NSTEPS)):
        in_cp[step] = start_in(step)

    for step in range(_NSTEPS):
        b = step % _NBUF
        in_cp[step].wait()
        out_cp[step] = pltpu.async_copy(
            buf.at[b],
            out_hbm.at[pl.ds(base + step * _CHUNK, _CHUNK)],
            sem_out.at[b],
        )
        prev = step - 1
        nxt = prev + _NBUF
        if prev >= 0 and nxt < _NSTEPS:
            out_cp[prev].wait()
            in_cp[nxt] = start_in(nxt)

    for step in range(max(0, _NSTEPS - _NBUF), _NSTEPS):
        out_cp[step].wait()


_TC_BLOCK = 512


def _tc_body(in_ref, out_ref):
    out_ref[...] = in_ref[...]


_tc_copy = pl.pallas_call(
    _tc_body,
    grid=(_TC_ROWS // _TC_BLOCK,),
    in_specs=[
        pl.BlockSpec(
            (_TC_BLOCK, MODEL_DIM), lambda i: (i + _SC_ROWS // _TC_BLOCK, 0)
        )
    ],
    out_specs=pl.BlockSpec(
        (_TC_BLOCK, MODEL_DIM), lambda i: (i + _SC_ROWS // _TC_BLOCK, 0)
    ),
    out_shape=jax.ShapeDtypeStruct((SEQ_LEN, MODEL_DIM), jnp.float32),
)


def kernel(x, emb_weight):
    del x
    sc_part = _sc_copy(emb_weight)
    tc_full = _tc_copy(emb_weight)
    return lax.dynamic_update_slice(tc_full, sc_part, (0, 0))


# speedup vs baseline: 2.4927x; 1.9516x over previous
import jax
import jax.numpy as jnp
from jax.experimental import pallas as pl

SEQ_LEN = 8192
MODEL_DIM = 1024
_TC_BLOCK = 512


def _tc_body(in_ref, out_ref):
    out_ref[...] = in_ref[...]


_tc_copy = pl.pallas_call(
    _tc_body,
    grid=(SEQ_LEN // _TC_BLOCK,),
    in_specs=[pl.BlockSpec((_TC_BLOCK, MODEL_DIM), lambda i: (i, 0))],
    out_specs=pl.BlockSpec((_TC_BLOCK, MODEL_DIM), lambda i: (i, 0)),
    out_shape=jax.ShapeDtypeStruct((SEQ_LEN, MODEL_DIM), jnp.float32),
)


def kernel(x, emb_weight):
    del x
    return _tc_copy(emb_weight)
